# baseline (speedup 1.0000x reference)
import functools

import jax
import jax.numpy as jnp
from jax import lax
from jax.experimental import pallas as pl
from jax.experimental.pallas import tpu as pltpu
from jax.experimental.pallas import tpu_sc as plsc

NUM_HEADS = 6
SCALE_FACTOR = 0.5
COEF_PRO = 0.3

_L = 16
_NC = 2
_NS = 16
_NW = _NC * _NS


def _ln(x, g, b, eps=1e-5):
    m = jnp.mean(x, axis=-1, keepdims=True)
    v = jnp.var(x, axis=-1, keepdims=True)
    return (x - m) / jnp.sqrt(v + eps) * g + b


def _bn_gelu(v, g, b, eps=1e-5):
    m = jnp.mean(v, axis=0)
    var = jnp.var(v, axis=0)
    y = (v - m) / jnp.sqrt(var + eps) * g + b
    return jax.nn.gelu(y, approximate=False)


def _cos(a, b):
    num = jnp.sum(a * b, axis=-1)
    den = jnp.linalg.norm(a, axis=-1) * jnp.linalg.norm(b, axis=-1)
    return num / jnp.maximum(den, 1e-8)


@functools.lru_cache(maxsize=None)
def _make_segred(Ntok, C, S):
    assert S % (_NW * _L) == 0 and C % _L == 0 and Ntok % _L == 0
    spw = S // _NW
    CH = C // _L
    NSCAN = Ntok // _L
    mesh = plsc.VectorSubcoreMesh(core_axis_name="c", subcore_axis_name="s")

    def body(tok_hbm, idx_hbm, neg_hbm, zero_hbm, out_hbm, idxv, mxv, smv, cntv, gbuf, semA, semB):
        wid = lax.axis_index("s") * _NC + lax.axis_index("c")
        base = wid * spw

        pltpu.sync_copy(idx_hbm, idxv)
        pltpu.sync_copy(neg_hbm, mxv)
        pltpu.sync_copy(zero_hbm, smv)

        zf = jnp.zeros((_L,), dtype=jnp.float32)
        onef = jnp.ones((_L,), dtype=jnp.float32)
        zeroi = jnp.zeros((_L,), dtype=jnp.int32)
        onei = jnp.ones((_L,), dtype=jnp.int32)
        lane = lax.iota(jnp.int32, _L)

        def init_cnt(s, _):
            cntv[s, :] = zf
            return 0
        lax.fori_loop(0, spw // _L, init_cnt, 0)

        basev = jnp.full((_L,), base, dtype=jnp.int32)
        topv = jnp.full((_L,), base + spw, dtype=jnp.int32)

        def probe(i):
            vec = idxv[pl.ds(i * _L, _L)]
            ms = jnp.where(vec >= basev, onei, zeroi) * jnp.where(vec < topv, onei, zeroi)
            hits = ms[0]
            for j in range(1, _L):
                hits = hits + ms[j]
            return vec, ms, hits

        def fire(i, slot, hits, sem_s):
            @pl.when(hits > 0)
            def _():
                pltpu.async_copy(tok_hbm.at[pl.ds(i * _L, _L)], gbuf.at[slot], sem_s)

        def drain_acc(i, slot, vec, ms, hits, sem_s):
            @pl.when(hits > 0)
            def _():
                pltpu.make_async_copy(tok_hbm.at[pl.ds(i * _L, _L)], gbuf.at[slot], sem_s).wait()
                for j in range(_L):
                    @pl.when(ms[j] > 0)
                    def _():
                        o = vec[j] - base
                        o4 = o // _L
                        olv = jnp.full((_L,), o % _L, dtype=jnp.int32)
                        cntv[o4, :] = cntv[o4, :] + jnp.where(lane == olv, onef, zf)

                        def t_body(t, _):
                            sl = pl.ds(t * _L, _L)
                            r = gbuf[slot, j, sl]
                            mxv[o, sl] = jnp.maximum(mxv[o, sl], r)
                            smv[o, sl] = smv[o, sl] + r
                            return 0
                        lax.fori_loop(0, CH, t_body, 0)

        vec0, ms0, hits0 = probe(0)
        fire(0, 0, hits0, semA)

        def pair_body(c, carry):
            vecA, msA, hitsA = carry
            iA = 2 * c
            iB = 2 * c + 1
            vecB, msB, hitsB = probe(iB)
            fire(iB, 1, hitsB, semB)
            drain_acc(iA, 0, vecA, msA, hitsA, semA)
            iA2 = iA + 2
            vecA2, msA2, hitsA2 = probe(iA2)

            @pl.when(iA2 < NSCAN)
            def _():
                fire(iA2, 0, hitsA2, semA)
            drain_acc(iB, 1, vecB, msB, hitsB, semB)
            return vecA2, msA2, hitsA2
        lax.fori_loop(0, NSCAN // 2, pair_body, (vec0, ms0, hits0))

        lowv = jnp.full((_L,), -1e30, dtype=jnp.float32)

        def fin_group(sg, _):
            crow = cntv[sg, :]
            invv = 1.0 / jnp.maximum(crow, onef)
            hasf = jnp.where(crow > zf, onef, zf)
            for j in range(_L):
                s = sg * _L + j
                cjv = jnp.full((_L,), invv[j], dtype=jnp.float32)
                hjv = jnp.full((_L,), hasf[j], dtype=jnp.float32)

                def fin_chunk(t, _):
                    sl = pl.ds(t * _L, _L)
                    mx = jnp.maximum(mxv[s, sl], lowv) * hjv
                    mxv[s, sl] = mx + smv[s, sl] * cjv
                    return 0
                lax.fori_loop(0, CH, fin_chunk, 0)
            return 0
        lax.fori_loop(0, spw // _L, fin_group, 0)

        pltpu.sync_copy(mxv, out_hbm.at[pl.ds(base, spw)])

    return pl.kernel(
        body,
        out_type=jax.ShapeDtypeStruct((S, C), jnp.float32),
        mesh=mesh,
        scratch_types=[
            pltpu.VMEM((Ntok,), jnp.int32),
            pltpu.VMEM((spw, C), jnp.float32),
            pltpu.VMEM((spw, C), jnp.float32),
            pltpu.VMEM((spw // _L, _L), jnp.float32),
            pltpu.VMEM((2, _L, C), jnp.float32),
            pltpu.SemaphoreType.DMA,
            pltpu.SemaphoreType.DMA,
        ],
        name=f"segred_{S}",
    )


def _seg_table(vals, seg_ids, num_segments):
    fn = _make_segred(vals.shape[0], vals.shape[1], num_segments)
    spw = num_segments // _NW
    neg = jnp.full((spw, vals.shape[1]), -jnp.inf, dtype=jnp.float32)
    zero = jnp.zeros((spw, vals.shape[1]), dtype=jnp.float32)
    return fn(vals, seg_ids.astype(jnp.int32), neg, zero)


def kernel(x, center1, mask, qkv_w, proj_w, proj_b, ls1_g, ls2_g, norm1_g, norm1_b, norm2_g, norm2_b, fc1_w, fc1_b, fc2_w, fc2_b, ad_down_w, ad_down_b, ad_up_w, ad_up_b, bn3d_g, bn3d_b, bn2d_g, bn2d_b, attn1_w, attn1_b, norm3_g, norm3_b, idx_ptr, sorted_cluster_indices, cluster, flat_grid_index, grid_shape):
    B, N, C = x.shape
    H = NUM_HEADS
    dh = C // H
    h = _ln(x, norm1_g, norm1_b)
    qkv = (h @ qkv_w.T).reshape(B, N, 3, H, dh).transpose(2, 0, 3, 1, 4)
    q, k, v = qkv[0], qkv[1], qkv[2]
    attn = jax.nn.softmax((q @ jnp.swapaxes(k, -2, -1)) * (dh ** -0.5), axis=-1)
    xa = (jnp.swapaxes(attn @ v, 1, 2).reshape(B, N, C)) @ proj_w.T + proj_b
    x = x + ls1_g * xa
    h2 = _ln(x, norm2_g, norm2_b)
    x_ffn = ls2_g * (jax.nn.gelu(h2 @ fc1_w.T + fc1_b, approximate=False) @ fc2_w.T + fc2_b)
    ad = jax.nn.gelu(x_ffn @ ad_down_w.T + ad_down_b, approximate=False) @ ad_up_w.T + ad_up_b
    x = x + x_ffn + SCALE_FACTOR * ad
    cls_x = x[:, 0]
    xb = x[:, 1:]
    feat = xb.reshape(-1, C)
    n_clusters = int(idx_ptr.shape[0]) - 1
    t3d = _seg_table(feat, cluster, n_clusters)
    x3d = _bn_gelu(t3d, bn3d_g, bn3d_b)[cluster].reshape(xb.shape)
    GS_STATIC = 16
    dim_size = int(xb.shape[0]) * GS_STATIC * GS_STATIC
    grid_shape_residual = grid_shape - GS_STATIC
    Vv = center1.shape[1]
    pospara = []
    for i in range(Vv):
        flat_x = xb.reshape(-1, C)
        a = (_ln(flat_x, norm3_g[i], norm3_b[i]) @ attn1_w[i].T + attn1_b[i]) * mask[i]
        flat_x = a + flat_x
        idx = flat_grid_index[i] + grid_shape_residual
        t2d = _seg_table(flat_x, idx, dim_size)
        z = _bn_gelu(t2d, bn2d_g[i], bn2d_b[i])
        pospara.append(z[idx].reshape(xb.shape))
    x_sup = jnp.swapaxes(jnp.stack(pospara, 0), 0, 1)
    sims = jnp.stack([(_cos(t, x3d) + 1.0) / 2.0 for t in pospara], 0)
    sims = jnp.swapaxes(sims, 0, 1)
    sims = sims / jnp.sum(sims, axis=1, keepdims=True)
    x_sup_w = jnp.sum(x_sup * sims[..., None], axis=1)
    xb = xb + COEF_PRO * x_sup_w
    out = jnp.concatenate([cls_x[:, None, :], xb], axis=1)
    return out, attn


# speedup vs baseline: 1.0229x; 1.0229x over previous
import functools

import jax
import jax.numpy as jnp
from jax import lax
from jax.experimental import pallas as pl
from jax.experimental.pallas import tpu as pltpu
from jax.experimental.pallas import tpu_sc as plsc

NUM_HEADS = 6
SCALE_FACTOR = 0.5
COEF_PRO = 0.3

_L = 16
_NC = 2
_NS = 16
_NW = _NC * _NS


def _ln(x, g, b, eps=1e-5):
    m = jnp.mean(x, axis=-1, keepdims=True)
    v = jnp.var(x, axis=-1, keepdims=True)
    return (x - m) / jnp.sqrt(v + eps) * g + b


def _bn_gelu(v, g, b, eps=1e-5):
    m = jnp.mean(v, axis=0)
    var = jnp.var(v, axis=0)
    y = (v - m) / jnp.sqrt(var + eps) * g + b
    return jax.nn.gelu(y, approximate=False)


def _cos(a, b):
    num = jnp.sum(a * b, axis=-1)
    den = jnp.linalg.norm(a, axis=-1) * jnp.linalg.norm(b, axis=-1)
    return num / jnp.maximum(den, 1e-8)


@functools.lru_cache(maxsize=None)
def _make_segred(Ntok, C, S):
    assert S % (_NW * _L) == 0 and C % _L == 0 and Ntok % _L == 0
    spw = S // _NW
    CH = C // _L
    NSCAN = Ntok // _L
    mesh = plsc.VectorSubcoreMesh(core_axis_name="c", subcore_axis_name="s")

    def body(tok_hbm, idx_hbm, neg_hbm, zero_hbm, out_hbm, idxv, mxv, smv, cntv, gbuf, sem):
        wid = lax.axis_index("s") * _NC + lax.axis_index("c")
        base = wid * spw

        pltpu.sync_copy(idx_hbm, idxv)
        pltpu.sync_copy(neg_hbm, mxv)
        pltpu.sync_copy(zero_hbm, smv)

        zf = jnp.zeros((_L,), dtype=jnp.float32)
        onef = jnp.ones((_L,), dtype=jnp.float32)
        zeroi = jnp.zeros((_L,), dtype=jnp.int32)
        onei = jnp.ones((_L,), dtype=jnp.int32)
        lane = lax.iota(jnp.int32, _L)

        def init_cnt(s, _):
            cntv[s, :] = zf
            return 0
        lax.fori_loop(0, spw // _L, init_cnt, 0)

        basev = jnp.full((_L,), base, dtype=jnp.int32)
        topv = jnp.full((_L,), base + spw, dtype=jnp.int32)

        def chunk_body(i, _):
            vec = idxv[pl.ds(i * _L, _L)]
            ms = jnp.where(vec >= basev, onei, zeroi) * jnp.where(vec < topv, onei, zeroi)
            hits = ms[0]
            for j in range(1, _L):
                hits = hits + ms[j]

            @pl.when(hits > 0)
            def _():
                pltpu.sync_copy(tok_hbm.at[pl.ds(i * _L, _L)], gbuf)
                for j in range(_L):
                    @pl.when(ms[j] > 0)
                    def _():
                        o = vec[j] - base
                        o4 = o // _L
                        olv = jnp.full((_L,), o % _L, dtype=jnp.int32)
                        cntv[o4, :] = cntv[o4, :] + jnp.where(lane == olv, onef, zf)

                        def t_body(t, _):
                            sl = pl.ds(t * _L, _L)
                            r = gbuf[j, sl]
                            mxv[o, sl] = jnp.maximum(mxv[o, sl], r)
                            smv[o, sl] = smv[o, sl] + r
                            return 0
                        lax.fori_loop(0, CH, t_body, 0)
            return 0
        lax.fori_loop(0, NSCAN, chunk_body, 0)

        lowv = jnp.full((_L,), -1e30, dtype=jnp.float32)

        def fin_group(sg, _):
            crow = cntv[sg, :]
            invv = 1.0 / jnp.maximum(crow, onef)
            hasf = jnp.where(crow > zf, onef, zf)
            for j in range(_L):
                s = sg * _L + j
                cjv = jnp.full((_L,), invv[j], dtype=jnp.float32)
                hjv = jnp.full((_L,), hasf[j], dtype=jnp.float32)

                def fin_chunk(t, _):
                    sl = pl.ds(t * _L, _L)
                    mx = jnp.maximum(mxv[s, sl], lowv) * hjv
                    mxv[s, sl] = mx + smv[s, sl] * cjv
                    return 0
                lax.fori_loop(0, CH, fin_chunk, 0)
            return 0
        lax.fori_loop(0, spw // _L, fin_group, 0)

        pltpu.sync_copy(mxv, out_hbm.at[pl.ds(base, spw)])

    return pl.kernel(
        body,
        out_type=jax.ShapeDtypeStruct((S, C), jnp.float32),
        mesh=mesh,
        scratch_types=[
            pltpu.VMEM((Ntok,), jnp.int32),
            pltpu.VMEM((spw, C), jnp.float32),
            pltpu.VMEM((spw, C), jnp.float32),
            pltpu.VMEM((spw // _L, _L), jnp.float32),
            pltpu.VMEM((_L, C), jnp.float32),
            pltpu.SemaphoreType.DMA,
        ],
        name=f"segred_{S}",
    )


@functools.lru_cache(maxsize=None)
def _make_gather(R, C, T):
    BR = 128
    rpw = R // _NW
    assert rpw % BR == 0 and C % _L == 0
    nch = rpw // BR
    mesh = plsc.VectorSubcoreMesh(core_axis_name="c", subcore_axis_name="s")

    def body(table_hbm, idx_hbm, out_hbm, idxv, gbuf, semA, semB):
        wid = lax.axis_index("s") * _NC + lax.axis_index("c")
        rbase = wid * rpw
        pltpu.sync_copy(idx_hbm.at[pl.ds(rbase, rpw)], idxv)

        pltpu.async_copy(table_hbm.at[idxv.at[pl.ds(0, BR)]], gbuf.at[0], semA)

        def chunk(k, _):
            par = k % 2
            sem_cur = semA

            @pl.when(k + 1 < nch)
            def _():
                @pl.when(par == 0)
                def _():
                    pltpu.async_copy(
                        table_hbm.at[idxv.at[pl.ds((k + 1) * BR, BR)]], gbuf.at[1], semB)

                @pl.when(par == 1)
                def _():
                    pltpu.async_copy(
                        table_hbm.at[idxv.at[pl.ds((k + 1) * BR, BR)]], gbuf.at[0], semA)

            @pl.when(par == 0)
            def _():
                pltpu.make_async_copy(
                    table_hbm.at[idxv.at[pl.ds(k * BR, BR)]], gbuf.at[0], semA).wait()
                pltpu.sync_copy(gbuf.at[0], out_hbm.at[pl.ds(rbase + k * BR, BR)])

            @pl.when(par == 1)
            def _():
                pltpu.make_async_copy(
                    table_hbm.at[idxv.at[pl.ds(k * BR, BR)]], gbuf.at[1], semB).wait()
                pltpu.sync_copy(gbuf.at[1], out_hbm.at[pl.ds(rbase + k * BR, BR)])
            return 0
        lax.fori_loop(0, nch, chunk, 0)

    return pl.kernel(
        body,
        out_type=jax.ShapeDtypeStruct((R, C), jnp.float32),
        mesh=mesh,
        scratch_types=[
            pltpu.VMEM((rpw,), jnp.int32),
            pltpu.VMEM((2, BR, C), jnp.float32),
            pltpu.SemaphoreType.DMA,
            pltpu.SemaphoreType.DMA,
        ],
        name="gather_rows",
    )


def _gather_rows(table, idx):
    fn = _make_gather(idx.shape[0], table.shape[1], table.shape[0])
    return fn(table, idx.astype(jnp.int32))


def _seg_table(vals, seg_ids, num_segments):
    fn = _make_segred(vals.shape[0], vals.shape[1], num_segments)
    spw = num_segments // _NW
    neg = jnp.full((spw, vals.shape[1]), -jnp.inf, dtype=jnp.float32)
    zero = jnp.zeros((spw, vals.shape[1]), dtype=jnp.float32)
    return fn(vals, seg_ids.astype(jnp.int32), neg, zero)


def kernel(x, center1, mask, qkv_w, proj_w, proj_b, ls1_g, ls2_g, norm1_g, norm1_b, norm2_g, norm2_b, fc1_w, fc1_b, fc2_w, fc2_b, ad_down_w, ad_down_b, ad_up_w, ad_up_b, bn3d_g, bn3d_b, bn2d_g, bn2d_b, attn1_w, attn1_b, norm3_g, norm3_b, idx_ptr, sorted_cluster_indices, cluster, flat_grid_index, grid_shape):
    B, N, C = x.shape
    H = NUM_HEADS
    dh = C // H
    h = _ln(x, norm1_g, norm1_b)
    qkv = (h @ qkv_w.T).reshape(B, N, 3, H, dh).transpose(2, 0, 3, 1, 4)
    q, k, v = qkv[0], qkv[1], qkv[2]
    attn = jax.nn.softmax((q @ jnp.swapaxes(k, -2, -1)) * (dh ** -0.5), axis=-1)
    xa = (jnp.swapaxes(attn @ v, 1, 2).reshape(B, N, C)) @ proj_w.T + proj_b
    x = x + ls1_g * xa
    h2 = _ln(x, norm2_g, norm2_b)
    x_ffn = ls2_g * (jax.nn.gelu(h2 @ fc1_w.T + fc1_b, approximate=False) @ fc2_w.T + fc2_b)
    ad = jax.nn.gelu(x_ffn @ ad_down_w.T + ad_down_b, approximate=False) @ ad_up_w.T + ad_up_b
    x = x + x_ffn + SCALE_FACTOR * ad
    cls_x = x[:, 0]
    xb = x[:, 1:]
    feat = xb.reshape(-1, C)
    n_clusters = int(idx_ptr.shape[0]) - 1
    t3d = _seg_table(feat, cluster, n_clusters)
    z3d = _bn_gelu(t3d, bn3d_g, bn3d_b)
    GS_STATIC = 16
    dim_size = int(xb.shape[0]) * GS_STATIC * GS_STATIC
    grid_shape_residual = grid_shape - GS_STATIC
    Vv = center1.shape[1]
    ztabs = [jnp.pad(z3d, ((0, dim_size - z3d.shape[0]), (0, 0)))]
    idxs = [cluster.astype(jnp.int32)]
    for i in range(Vv):
        flat_x = xb.reshape(-1, C)
        a = (_ln(flat_x, norm3_g[i], norm3_b[i]) @ attn1_w[i].T + attn1_b[i]) * mask[i]
        flat_x = a + flat_x
        idx = flat_grid_index[i] + grid_shape_residual
        t2d = _seg_table(flat_x, idx, dim_size)
        ztabs.append(_bn_gelu(t2d, bn2d_g[i], bn2d_b[i]))
        idxs.append((idx + (i + 1) * dim_size).astype(jnp.int32))
    table = jnp.concatenate(ztabs, axis=0)
    flat_idx = jnp.concatenate(idxs, axis=0)
    gathered = _gather_rows(table, flat_idx)
    Ntok = xb.shape[0] * xb.shape[1]
    x3d = gathered[:Ntok].reshape(xb.shape)
    pospara = [gathered[(i + 1) * Ntok:(i + 2) * Ntok].reshape(xb.shape) for i in range(Vv)]
    x_sup = jnp.swapaxes(jnp.stack(pospara, 0), 0, 1)
    sims = jnp.stack([(_cos(t, x3d) + 1.0) / 2.0 for t in pospara], 0)
    sims = jnp.swapaxes(sims, 0, 1)
    sims = sims / jnp.sum(sims, axis=1, keepdims=True)
    x_sup_w = jnp.sum(x_sup * sims[..., None], axis=1)
    xb = xb + COEF_PRO * x_sup_w
    out = jnp.concatenate([cls_x[:, None, :], xb], axis=1)
    return out, attn


# speedup vs baseline: 1.1062x; 1.0813x over previous
import functools

import jax
import jax.numpy as jnp
from jax import lax
from jax.experimental import pallas as pl
from jax.experimental.pallas import tpu as pltpu
from jax.experimental.pallas import tpu_sc as plsc

NUM_HEADS = 6
SCALE_FACTOR = 0.5
COEF_PRO = 0.3

_L = 16
_NC = 2
_NS = 16
_NW = _NC * _NS


def _ln(x, g, b, eps=1e-5):
    m = jnp.mean(x, axis=-1, keepdims=True)
    v = jnp.var(x, axis=-1, keepdims=True)
    return (x - m) / jnp.sqrt(v + eps) * g + b


def _bn_gelu(v, g, b, eps=1e-5):
    m = jnp.mean(v, axis=0)
    var = jnp.var(v, axis=0)
    y = (v - m) / jnp.sqrt(var + eps) * g + b
    return jax.nn.gelu(y, approximate=False)


def _cos(a, b):
    num = jnp.sum(a * b, axis=-1)
    den = jnp.linalg.norm(a, axis=-1) * jnp.linalg.norm(b, axis=-1)
    return num / jnp.maximum(den, 1e-8)


def _attn_body(q_ref, k_ref, v_ref, attn_ref, o_ref):
    q = q_ref[0, 0]
    k = k_ref[0, 0]
    v = v_ref[0, 0]
    dh = q.shape[-1]
    s = jax.lax.dot_general(q, k, (((1,), (1,)), ((), ())),
                            preferred_element_type=jnp.float32) * (dh ** -0.5)
    m = jnp.max(s, axis=-1, keepdims=True)
    e = jnp.exp(s - m)
    a = e / jnp.sum(e, axis=-1, keepdims=True)
    attn_ref[0, 0] = a
    o_ref[0, 0] = jnp.dot(a, v, preferred_element_type=jnp.float32)


@functools.lru_cache(maxsize=None)
def _make_attn(B, H, N, dh):
    return pl.pallas_call(
        _attn_body,
        grid=(B, H),
        in_specs=[
            pl.BlockSpec((1, 1, N, dh), lambda b, h: (b, h, 0, 0)),
            pl.BlockSpec((1, 1, N, dh), lambda b, h: (b, h, 0, 0)),
            pl.BlockSpec((1, 1, N, dh), lambda b, h: (b, h, 0, 0)),
        ],
        out_specs=[
            pl.BlockSpec((1, 1, N, N), lambda b, h: (b, h, 0, 0)),
            pl.BlockSpec((1, 1, N, dh), lambda b, h: (b, h, 0, 0)),
        ],
        out_shape=[
            jax.ShapeDtypeStruct((B, H, N, N), jnp.float32),
            jax.ShapeDtypeStruct((B, H, N, dh), jnp.float32),
        ],
        name="mha_attn",
    )


@functools.lru_cache(maxsize=None)
def _make_segred(Ntok, C, S):
    assert S % (_NW * _L) == 0 and C % _L == 0 and Ntok % _L == 0
    spw = S // _NW
    CH = C // _L
    NSCAN = Ntok // _L
    mesh = plsc.VectorSubcoreMesh(core_axis_name="c", subcore_axis_name="s")

    def body(tok_hbm, idx_hbm, neg_hbm, zero_hbm, out_hbm, idxv, mxv, smv, cntv, gbuf, sem):
        wid = lax.axis_index("s") * _NC + lax.axis_index("c")
        base = wid * spw

        pltpu.sync_copy(idx_hbm, idxv)
        pltpu.sync_copy(neg_hbm, mxv)
        pltpu.sync_copy(zero_hbm, smv)

        zf = jnp.zeros((_L,), dtype=jnp.float32)
        onef = jnp.ones((_L,), dtype=jnp.float32)
        zeroi = jnp.zeros((_L,), dtype=jnp.int32)
        onei = jnp.ones((_L,), dtype=jnp.int32)
        lane = lax.iota(jnp.int32, _L)

        def init_cnt(s, _):
            cntv[s, :] = zf
            return 0
        lax.fori_loop(0, spw // _L, init_cnt, 0)

        basev = jnp.full((_L,), base, dtype=jnp.int32)
        topv = jnp.full((_L,), base + spw, dtype=jnp.int32)

        def chunk_body(i, _):
            vec = idxv[pl.ds(i * _L, _L)]
            ms = jnp.where(vec >= basev, onei, zeroi) * jnp.where(vec < topv, onei, zeroi)
            hits = ms[0]
            for j in range(1, _L):
                hits = hits + ms[j]

            @pl.when(hits > 0)
            def _():
                pltpu.sync_copy(tok_hbm.at[pl.ds(i * _L, _L)], gbuf)
                for j in range(_L):
                    @pl.when(ms[j] > 0)
                    def _():
                        o = vec[j] - base
                        o4 = o // _L
                        olv = jnp.full((_L,), o % _L, dtype=jnp.int32)
                        cntv[o4, :] = cntv[o4, :] + jnp.where(lane == olv, onef, zf)

                        def t_body(t, _):
                            sl = pl.ds(t * _L, _L)
                            r = gbuf[j, sl]
                            mxv[o, sl] = jnp.maximum(mxv[o, sl], r)
                            smv[o, sl] = smv[o, sl] + r
                            return 0
                        lax.fori_loop(0, CH, t_body, 0)
            return 0
        lax.fori_loop(0, NSCAN, chunk_body, 0)

        lowv = jnp.full((_L,), -1e30, dtype=jnp.float32)

        def fin_group(sg, _):
            crow = cntv[sg, :]
            invv = 1.0 / jnp.maximum(crow, onef)
            hasf = jnp.where(crow > zf, onef, zf)
            for j in range(_L):
                s = sg * _L + j
                cjv = jnp.full((_L,), invv[j], dtype=jnp.float32)
                hjv = jnp.full((_L,), hasf[j], dtype=jnp.float32)

                def fin_chunk(t, _):
                    sl = pl.ds(t * _L, _L)
                    mx = jnp.maximum(mxv[s, sl], lowv) * hjv
                    mxv[s, sl] = mx + smv[s, sl] * cjv
                    return 0
                lax.fori_loop(0, CH, fin_chunk, 0)
            return 0
        lax.fori_loop(0, spw // _L, fin_group, 0)

        pltpu.sync_copy(mxv, out_hbm.at[pl.ds(base, spw)])

    return pl.kernel(
        body,
        out_type=jax.ShapeDtypeStruct((S, C), jnp.float32),
        mesh=mesh,
        scratch_types=[
            pltpu.VMEM((Ntok,), jnp.int32),
            pltpu.VMEM((spw, C), jnp.float32),
            pltpu.VMEM((spw, C), jnp.float32),
            pltpu.VMEM((spw // _L, _L), jnp.float32),
            pltpu.VMEM((_L, C), jnp.float32),
            pltpu.SemaphoreType.DMA,
        ],
        name=f"segred_{S}",
    )


@functools.lru_cache(maxsize=None)
def _make_gather(R, C, T):
    BR = 128
    rpw = R // _NW
    assert rpw % BR == 0 and C % _L == 0
    nch = rpw // BR
    mesh = plsc.VectorSubcoreMesh(core_axis_name="c", subcore_axis_name="s")

    def body(table_hbm, idx_hbm, out_hbm, idxv, gbuf, semA, semB):
        wid = lax.axis_index("s") * _NC + lax.axis_index("c")
        rbase = wid * rpw
        pltpu.sync_copy(idx_hbm.at[pl.ds(rbase, rpw)], idxv)

        pltpu.async_copy(table_hbm.at[idxv.at[pl.ds(0, BR)]], gbuf.at[0], semA)

        def chunk(k, _):
            par = k % 2
            sem_cur = semA

            @pl.when(k + 1 < nch)
            def _():
                @pl.when(par == 0)
                def _():
                    pltpu.async_copy(
                        table_hbm.at[idxv.at[pl.ds((k + 1) * BR, BR)]], gbuf.at[1], semB)

                @pl.when(par == 1)
                def _():
                    pltpu.async_copy(
                        table_hbm.at[idxv.at[pl.ds((k + 1) * BR, BR)]], gbuf.at[0], semA)

            @pl.when(par == 0)
            def _():
                pltpu.make_async_copy(
                    table_hbm.at[idxv.at[pl.ds(k * BR, BR)]], gbuf.at[0], semA).wait()
                pltpu.sync_copy(gbuf.at[0], out_hbm.at[pl.ds(rbase + k * BR, BR)])

            @pl.when(par == 1)
            def _():
                pltpu.make_async_copy(
                    table_hbm.at[idxv.at[pl.ds(k * BR, BR)]], gbuf.at[1], semB).wait()
                pltpu.sync_copy(gbuf.at[1], out_hbm.at[pl.ds(rbase + k * BR, BR)])
            return 0
        lax.fori_loop(0, nch, chunk, 0)

    return pl.kernel(
        body,
        out_type=jax.ShapeDtypeStruct((R, C), jnp.float32),
        mesh=mesh,
        scratch_types=[
            pltpu.VMEM((rpw,), jnp.int32),
            pltpu.VMEM((2, BR, C), jnp.float32),
            pltpu.SemaphoreType.DMA,
            pltpu.SemaphoreType.DMA,
        ],
        name="gather_rows",
    )


def _gather_rows(table, idx):
    fn = _make_gather(idx.shape[0], table.shape[1], table.shape[0])
    return fn(table, idx.astype(jnp.int32))


def _seg_table(vals, seg_ids, num_segments):
    fn = _make_segred(vals.shape[0], vals.shape[1], num_segments)
    spw = num_segments // _NW
    neg = jnp.full((spw, vals.shape[1]), -jnp.inf, dtype=jnp.float32)
    zero = jnp.zeros((spw, vals.shape[1]), dtype=jnp.float32)
    return fn(vals, seg_ids.astype(jnp.int32), neg, zero)


def kernel(x, center1, mask, qkv_w, proj_w, proj_b, ls1_g, ls2_g, norm1_g, norm1_b, norm2_g, norm2_b, fc1_w, fc1_b, fc2_w, fc2_b, ad_down_w, ad_down_b, ad_up_w, ad_up_b, bn3d_g, bn3d_b, bn2d_g, bn2d_b, attn1_w, attn1_b, norm3_g, norm3_b, idx_ptr, sorted_cluster_indices, cluster, flat_grid_index, grid_shape):
    B, N, C = x.shape
    H = NUM_HEADS
    dh = C // H
    h = _ln(x, norm1_g, norm1_b)
    qkv = (h @ qkv_w.T).reshape(B, N, 3, H, dh).transpose(2, 0, 3, 1, 4)
    q, k, v = qkv[0], qkv[1], qkv[2]
    attn, o = _make_attn(B, H, N, dh)(q, k, v)
    xa = (jnp.swapaxes(o, 1, 2).reshape(B, N, C)) @ proj_w.T + proj_b
    x = x + ls1_g * xa
    h2 = _ln(x, norm2_g, norm2_b)
    x_ffn = ls2_g * (jax.nn.gelu(h2 @ fc1_w.T + fc1_b, approximate=False) @ fc2_w.T + fc2_b)
    ad = jax.nn.gelu(x_ffn @ ad_down_w.T + ad_down_b, approximate=False) @ ad_up_w.T + ad_up_b
    x = x + x_ffn + SCALE_FACTOR * ad
    cls_x = x[:, 0]
    xb = x[:, 1:]
    feat = xb.reshape(-1, C)
    n_clusters = int(idx_ptr.shape[0]) - 1
    t3d = _seg_table(feat, cluster, n_clusters)
    z3d = _bn_gelu(t3d, bn3d_g, bn3d_b)
    GS_STATIC = 16
    dim_size = int(xb.shape[0]) * GS_STATIC * GS_STATIC
    grid_shape_residual = grid_shape - GS_STATIC
    Vv = center1.shape[1]
    ztabs = [jnp.pad(z3d, ((0, dim_size - z3d.shape[0]), (0, 0)))]
    idxs = [cluster.astype(jnp.int32)]
    for i in range(Vv):
        flat_x = xb.reshape(-1, C)
        a = (_ln(flat_x, norm3_g[i], norm3_b[i]) @ attn1_w[i].T + attn1_b[i]) * mask[i]
        flat_x = a + flat_x
        idx = flat_grid_index[i] + grid_shape_residual
        t2d = _seg_table(flat_x, idx, dim_size)
        ztabs.append(_bn_gelu(t2d, bn2d_g[i], bn2d_b[i]))
        idxs.append((idx + (i + 1) * dim_size).astype(jnp.int32))
    table = jnp.concatenate(ztabs, axis=0)
    flat_idx = jnp.concatenate(idxs, axis=0)
    gathered = _gather_rows(table, flat_idx)
    Ntok = xb.shape[0] * xb.shape[1]
    x3d = gathered[:Ntok].reshape(xb.shape)
    pospara = [gathered[(i + 1) * Ntok:(i + 2) * Ntok].reshape(xb.shape) for i in range(Vv)]
    x_sup = jnp.swapaxes(jnp.stack(pospara, 0), 0, 1)
    sims = jnp.stack([(_cos(t, x3d) + 1.0) / 2.0 for t in pospara], 0)
    sims = jnp.swapaxes(sims, 0, 1)
    sims = sims / jnp.sum(sims, axis=1, keepdims=True)
    x_sup_w = jnp.sum(x_sup * sims[..., None], axis=1)
    xb = xb + COEF_PRO * x_sup_w
    out = jnp.concatenate([cls_x[:, None, :], xb], axis=1)
    return out, attn


# speedup vs baseline: 1.1732x; 1.0606x over previous
import functools

import jax
import jax.numpy as jnp
from jax import lax
from jax.experimental import pallas as pl
from jax.experimental.pallas import tpu as pltpu
from jax.experimental.pallas import tpu_sc as plsc

NUM_HEADS = 6
SCALE_FACTOR = 0.5
COEF_PRO = 0.3

_L = 16
_NC = 2
_NS = 16
_NW = _NC * _NS


def _ln(x, g, b, eps=1e-5):
    m = jnp.mean(x, axis=-1, keepdims=True)
    v = jnp.var(x, axis=-1, keepdims=True)
    return (x - m) / jnp.sqrt(v + eps) * g + b


def _bn_gelu(v, g, b, eps=1e-5):
    m = jnp.mean(v, axis=0)
    var = jnp.var(v, axis=0)
    y = (v - m) / jnp.sqrt(var + eps) * g + b
    return jax.nn.gelu(y, approximate=False)


def _cos(a, b):
    num = jnp.sum(a * b, axis=-1)
    den = jnp.linalg.norm(a, axis=-1) * jnp.linalg.norm(b, axis=-1)
    return num / jnp.maximum(den, 1e-8)


def _attn_body(q_ref, k_ref, v_ref, attn_ref, o_ref):
    q = q_ref[0, 0]
    k = k_ref[0, 0]
    v = v_ref[0, 0]
    dh = q.shape[-1]
    s = jax.lax.dot_general(q, k, (((1,), (1,)), ((), ())),
                            preferred_element_type=jnp.float32) * (dh ** -0.5)
    m = jnp.max(s, axis=-1, keepdims=True)
    e = jnp.exp(s - m)
    a = e / jnp.sum(e, axis=-1, keepdims=True)
    attn_ref[0, 0] = a
    o_ref[0, 0] = jnp.dot(a, v, preferred_element_type=jnp.float32)


@functools.lru_cache(maxsize=None)
def _make_attn(B, H, N, dh):
    return pl.pallas_call(
        _attn_body,
        grid=(B, H),
        in_specs=[
            pl.BlockSpec((1, 1, N, dh), lambda b, h: (b, h, 0, 0)),
            pl.BlockSpec((1, 1, N, dh), lambda b, h: (b, h, 0, 0)),
            pl.BlockSpec((1, 1, N, dh), lambda b, h: (b, h, 0, 0)),
        ],
        out_specs=[
            pl.BlockSpec((1, 1, N, N), lambda b, h: (b, h, 0, 0)),
            pl.BlockSpec((1, 1, N, dh), lambda b, h: (b, h, 0, 0)),
        ],
        out_shape=[
            jax.ShapeDtypeStruct((B, H, N, N), jnp.float32),
            jax.ShapeDtypeStruct((B, H, N, dh), jnp.float32),
        ],
        name="mha_attn",
    )


def _gelu_exact(x):
    return 0.5 * x * (1.0 + jax.lax.erf(x * 0.7071067811865476))


def _ffn_body(xin_ref, om_ref, pw_ref, pb_ref, ls1_ref, g_ref, b_ref, ls2_ref,
              w1_ref, b1_ref, w2_ref, b2_ref, wd_ref, bd_ref, wu_ref, bu_ref, out_ref):
    xin = xin_ref[0]
    om = om_ref[0]
    dims = (((1,), (1,)), ((), ()))
    xa = jax.lax.dot_general(om, pw_ref[...], dims,
                             preferred_element_type=jnp.float32) + pb_ref[...]
    x = xin + ls1_ref[...] * xa
    m = jnp.mean(x, axis=-1, keepdims=True)
    v = jnp.mean((x - m) ** 2, axis=-1, keepdims=True)
    h2 = (x - m) / jnp.sqrt(v + 1e-5) * g_ref[...] + b_ref[...]
    f = _gelu_exact(
        jax.lax.dot_general(h2, w1_ref[...], dims,
                            preferred_element_type=jnp.float32) + b1_ref[...])
    xf = ls2_ref[...] * (
        jax.lax.dot_general(f, w2_ref[...], dims,
                            preferred_element_type=jnp.float32) + b2_ref[...])
    adh = _gelu_exact(
        jax.lax.dot_general(xf, wd_ref[...], dims,
                            preferred_element_type=jnp.float32) + bd_ref[...])
    ad = jax.lax.dot_general(adh, wu_ref[...], dims,
                             preferred_element_type=jnp.float32) + bu_ref[...]
    out_ref[0] = x + xf + SCALE_FACTOR * ad


@functools.lru_cache(maxsize=None)
def _make_ffn(B, N, C, HID, AD):
    full = lambda *shape: pl.BlockSpec(shape, lambda b: (0,) * len(shape))
    return pl.pallas_call(
        _ffn_body,
        grid=(B,),
        in_specs=[
            pl.BlockSpec((1, N, C), lambda b: (b, 0, 0)),
            pl.BlockSpec((1, N, C), lambda b: (b, 0, 0)),
            full(C, C), full(C), full(C), full(C), full(C), full(C),
            full(HID, C), full(HID), full(C, HID), full(C),
            full(AD, C), full(AD), full(C, AD), full(C),
        ],
        out_specs=pl.BlockSpec((1, N, C), lambda b: (b, 0, 0)),
        out_shape=jax.ShapeDtypeStruct((B, N, C), jnp.float32),
        name="proj_ffn_adapter",
    )


@functools.lru_cache(maxsize=None)
def _make_segred(Ntok, C, S):
    assert S % (_NW * _L) == 0 and C % _L == 0 and Ntok % _L == 0
    spw = S // _NW
    CH = C // _L
    NSCAN = Ntok // _L
    mesh = plsc.VectorSubcoreMesh(core_axis_name="c", subcore_axis_name="s")

    def body(tok_hbm, idx_hbm, neg_hbm, zero_hbm, out_hbm, idxv, mxv, smv, cntv, gbuf, sem):
        wid = lax.axis_index("s") * _NC + lax.axis_index("c")
        base = wid * spw

        pltpu.sync_copy(idx_hbm, idxv)
        pltpu.sync_copy(neg_hbm, mxv)
        pltpu.sync_copy(zero_hbm, smv)

        zf = jnp.zeros((_L,), dtype=jnp.float32)
        onef = jnp.ones((_L,), dtype=jnp.float32)
        zeroi = jnp.zeros((_L,), dtype=jnp.int32)
        onei = jnp.ones((_L,), dtype=jnp.int32)
        lane = lax.iota(jnp.int32, _L)

        def init_cnt(s, _):
            cntv[s, :] = zf
            return 0
        lax.fori_loop(0, spw // _L, init_cnt, 0)

        basev = jnp.full((_L,), base, dtype=jnp.int32)
        topv = jnp.full((_L,), base + spw, dtype=jnp.int32)

        def chunk_body(i, _):
            vec = idxv[pl.ds(i * _L, _L)]
            ms = jnp.where(vec >= basev, onei, zeroi) * jnp.where(vec < topv, onei, zeroi)
            hits = ms[0]
            for j in range(1, _L):
                hits = hits + ms[j]

            @pl.when(hits > 0)
            def _():
                pltpu.sync_copy(tok_hbm.at[pl.ds(i * _L, _L)], gbuf)
                for j in range(_L):
                    @pl.when(ms[j] > 0)
                    def _():
                        o = vec[j] - base
                        o4 = o // _L
                        olv = jnp.full((_L,), o % _L, dtype=jnp.int32)
                        cntv[o4, :] = cntv[o4, :] + jnp.where(lane == olv, onef, zf)

                        def t_body(t, _):
                            sl = pl.ds(t * _L, _L)
                            r = gbuf[j, sl]
                            mxv[o, sl] = jnp.maximum(mxv[o, sl], r)
                            smv[o, sl] = smv[o, sl] + r
                            return 0
                        lax.fori_loop(0, CH, t_body, 0)
            return 0
        lax.fori_loop(0, NSCAN, chunk_body, 0)

        lowv = jnp.full((_L,), -1e30, dtype=jnp.float32)

        def fin_group(sg, _):
            crow = cntv[sg, :]
            invv = 1.0 / jnp.maximum(crow, onef)
            hasf = jnp.where(crow > zf, onef, zf)
            for j in range(_L):
                s = sg * _L + j
                cjv = jnp.full((_L,), invv[j], dtype=jnp.float32)
                hjv = jnp.full((_L,), hasf[j], dtype=jnp.float32)

                def fin_chunk(t, _):
                    sl = pl.ds(t * _L, _L)
                    mx = jnp.maximum(mxv[s, sl], lowv) * hjv
                    mxv[s, sl] = mx + smv[s, sl] * cjv
                    return 0
                lax.fori_loop(0, CH, fin_chunk, 0)
            return 0
        lax.fori_loop(0, spw // _L, fin_group, 0)

        pltpu.sync_copy(mxv, out_hbm.at[pl.ds(base, spw)])

    return pl.kernel(
        body,
        out_type=jax.ShapeDtypeStruct((S, C), jnp.float32),
        mesh=mesh,
        scratch_types=[
            pltpu.VMEM((Ntok,), jnp.int32),
            pltpu.VMEM((spw, C), jnp.float32),
            pltpu.VMEM((spw, C), jnp.float32),
            pltpu.VMEM((spw // _L, _L), jnp.float32),
            pltpu.VMEM((_L, C), jnp.float32),
            pltpu.SemaphoreType.DMA,
        ],
        name=f"segred_{S}",
    )


@functools.lru_cache(maxsize=None)
def _make_gather(R, C, T):
    BR = 128
    rpw = R // _NW
    assert rpw % BR == 0 and C % _L == 0
    nch = rpw // BR
    mesh = plsc.VectorSubcoreMesh(core_axis_name="c", subcore_axis_name="s")

    def body(table_hbm, idx_hbm, out_hbm, idxv, gbuf, semA, semB):
        wid = lax.axis_index("s") * _NC + lax.axis_index("c")
        rbase = wid * rpw
        pltpu.sync_copy(idx_hbm.at[pl.ds(rbase, rpw)], idxv)

        pltpu.async_copy(table_hbm.at[idxv.at[pl.ds(0, BR)]], gbuf.at[0], semA)

        def chunk(k, _):
            par = k % 2
            sem_cur = semA

            @pl.when(k + 1 < nch)
            def _():
                @pl.when(par == 0)
                def _():
                    pltpu.async_copy(
                        table_hbm.at[idxv.at[pl.ds((k + 1) * BR, BR)]], gbuf.at[1], semB)

                @pl.when(par == 1)
                def _():
                    pltpu.async_copy(
                        table_hbm.at[idxv.at[pl.ds((k + 1) * BR, BR)]], gbuf.at[0], semA)

            @pl.when(par == 0)
            def _():
                pltpu.make_async_copy(
                    table_hbm.at[idxv.at[pl.ds(k * BR, BR)]], gbuf.at[0], semA).wait()
                pltpu.sync_copy(gbuf.at[0], out_hbm.at[pl.ds(rbase + k * BR, BR)])

            @pl.when(par == 1)
            def _():
                pltpu.make_async_copy(
                    table_hbm.at[idxv.at[pl.ds(k * BR, BR)]], gbuf.at[1], semB).wait()
                pltpu.sync_copy(gbuf.at[1], out_hbm.at[pl.ds(rbase + k * BR, BR)])
            return 0
        lax.fori_loop(0, nch, chunk, 0)

    return pl.kernel(
        body,
        out_type=jax.ShapeDtypeStruct((R, C), jnp.float32),
        mesh=mesh,
        scratch_types=[
            pltpu.VMEM((rpw,), jnp.int32),
            pltpu.VMEM((2, BR, C), jnp.float32),
            pltpu.SemaphoreType.DMA,
            pltpu.SemaphoreType.DMA,
        ],
        name="gather_rows",
    )


def _gather_rows(table, idx):
    fn = _make_gather(idx.shape[0], table.shape[1], table.shape[0])
    return fn(table, idx.astype(jnp.int32))


def _seg_table(vals, seg_ids, num_segments):
    fn = _make_segred(vals.shape[0], vals.shape[1], num_segments)
    spw = num_segments // _NW
    neg = jnp.full((spw, vals.shape[1]), -jnp.inf, dtype=jnp.float32)
    zero = jnp.zeros((spw, vals.shape[1]), dtype=jnp.float32)
    return fn(vals, seg_ids.astype(jnp.int32), neg, zero)


def kernel(x, center1, mask, qkv_w, proj_w, proj_b, ls1_g, ls2_g, norm1_g, norm1_b, norm2_g, norm2_b, fc1_w, fc1_b, fc2_w, fc2_b, ad_down_w, ad_down_b, ad_up_w, ad_up_b, bn3d_g, bn3d_b, bn2d_g, bn2d_b, attn1_w, attn1_b, norm3_g, norm3_b, idx_ptr, sorted_cluster_indices, cluster, flat_grid_index, grid_shape):
    B, N, C = x.shape
    H = NUM_HEADS
    dh = C // H
    h = _ln(x, norm1_g, norm1_b)
    qkv = (h @ qkv_w.T).reshape(B, N, 3, H, dh).transpose(2, 0, 3, 1, 4)
    q, k, v = qkv[0], qkv[1], qkv[2]
    attn, o = _make_attn(B, H, N, dh)(q, k, v)
    om = jnp.swapaxes(o, 1, 2).reshape(B, N, C)
    x = _make_ffn(B, N, C, fc1_w.shape[0], ad_down_w.shape[0])(
        x, om, proj_w, proj_b, ls1_g, norm2_g, norm2_b, ls2_g,
        fc1_w, fc1_b, fc2_w, fc2_b, ad_down_w, ad_down_b, ad_up_w, ad_up_b)
    cls_x = x[:, 0]
    xb = x[:, 1:]
    feat = xb.reshape(-1, C)
    n_clusters = int(idx_ptr.shape[0]) - 1
    t3d = _seg_table(feat, cluster, n_clusters)
    z3d = _bn_gelu(t3d, bn3d_g, bn3d_b)
    GS_STATIC = 16
    dim_size = int(xb.shape[0]) * GS_STATIC * GS_STATIC
    grid_shape_residual = grid_shape - GS_STATIC
    Vv = center1.shape[1]
    ztabs = [jnp.pad(z3d, ((0, dim_size - z3d.shape[0]), (0, 0)))]
    idxs = [cluster.astype(jnp.int32)]
    for i in range(Vv):
        flat_x = xb.reshape(-1, C)
        a = (_ln(flat_x, norm3_g[i], norm3_b[i]) @ attn1_w[i].T + attn1_b[i]) * mask[i]
        flat_x = a + flat_x
        idx = flat_grid_index[i] + grid_shape_residual
        t2d = _seg_table(flat_x, idx, dim_size)
        ztabs.append(_bn_gelu(t2d, bn2d_g[i], bn2d_b[i]))
        idxs.append((idx + (i + 1) * dim_size).astype(jnp.int32))
    table = jnp.concatenate(ztabs, axis=0)
    flat_idx = jnp.concatenate(idxs, axis=0)
    gathered = _gather_rows(table, flat_idx)
    Ntok = xb.shape[0] * xb.shape[1]
    x3d = gathered[:Ntok].reshape(xb.shape)
    pospara = [gathered[(i + 1) * Ntok:(i + 2) * Ntok].reshape(xb.shape) for i in range(Vv)]
    x_sup = jnp.swapaxes(jnp.stack(pospara, 0), 0, 1)
    sims = jnp.stack([(_cos(t, x3d) + 1.0) / 2.0 for t in pospara], 0)
    sims = jnp.swapaxes(sims, 0, 1)
    sims = sims / jnp.sum(sims, axis=1, keepdims=True)
    x_sup_w = jnp.sum(x_sup * sims[..., None], axis=1)
    xb = xb + COEF_PRO * x_sup_w
    out = jnp.concatenate([cls_x[:, None, :], xb], axis=1)
    return out, attn
